# baseline (speedup 1.0000x reference)
import functools

import jax
import jax.numpy as jnp
from jax import lax
from jax.experimental import pallas as pl
from jax.experimental.pallas import tpu as pltpu
from jax.experimental.pallas import tpu_sc as plsc

_N = 10000
_E = 320000
_F = 128
_L = 3
_EPS = 1e-5

_NC = 2
_NS = 16
_CHUNK = 128
_CH_PER_SUB = 80
_CH_PER_CORE = _CH_PER_SUB * _NS
_EPAD = _CH_PER_CORE * _NC * _CHUNK
_NLAND = 128
_NPAD = _N + _NLAND
_RPS = 624
_RPS_LAST = _N - 15 * _RPS

_BLK = 1000
_NBLK = _N // _BLK

_mesh = plsc.VectorSubcoreMesh(core_axis_name="c", subcore_axis_name="s",
                               num_cores=_NC, num_subcores=_NS)


def _sc_scatter_body(hp_hbm, srcp_hbm, dstp_hbm, out_hbm,
                     src_all, dst0, dst1, rows0, rows1, acc_sh,
                     sg0, sg1, sd0, sd1, ss0, ss1):
    c = lax.axis_index("c")
    s = lax.axis_index("s")
    r0 = s * _RPS

    @pl.when(s < _NS - 1)
    def _():
        pltpu.sync_copy(hp_hbm.at[pl.ds(r0, _RPS)],
                        acc_sh.at[pl.ds(r0, _RPS)])

    @pl.when(s == _NS - 1)
    def _():
        pltpu.sync_copy(hp_hbm.at[pl.ds(r0, _RPS_LAST)],
                        acc_sh.at[pl.ds(r0, _RPS_LAST)])

    brow = c * _CH_PER_CORE + s * _CH_PER_SUB
    pltpu.sync_copy(srcp_hbm.at[pl.ds(brow, _CH_PER_SUB)], src_all)

    pltpu.async_copy(hp_hbm.at[src_all.at[0]], rows0, sg0)
    pltpu.async_copy(hp_hbm.at[src_all.at[1]], rows1, sg1)
    pltpu.async_copy(dstp_hbm.at[brow], dst0, sd0)
    pltpu.async_copy(dstp_hbm.at[brow + 1], dst1, sd1)
    plsc.subcore_barrier()

    @pl.loop(0, _CH_PER_SUB // 2)
    def _(i):
        t0 = 2 * i
        pltpu.make_async_copy(hp_hbm.at[src_all.at[t0]], rows0, sg0).wait()
        pltpu.make_async_copy(dstp_hbm.at[brow + t0], dst0, sd0).wait()
        pltpu.async_copy(rows0, acc_sh.at[dst0], ss0, add=True)
        pltpu.make_async_copy(hp_hbm.at[src_all.at[t0 + 1]], rows1,
                              sg1).wait()
        pltpu.make_async_copy(dstp_hbm.at[brow + t0 + 1], dst1, sd1).wait()
        pltpu.async_copy(rows1, acc_sh.at[dst1], ss1, add=True)

        @pl.when(i < _CH_PER_SUB // 2 - 1)
        def _():
            pltpu.make_async_copy(rows0, acc_sh.at[dst0], ss0).wait()
            pltpu.async_copy(hp_hbm.at[src_all.at[t0 + 2]], rows0, sg0)
            pltpu.async_copy(dstp_hbm.at[brow + t0 + 2], dst0, sd0)
            pltpu.make_async_copy(rows1, acc_sh.at[dst1], ss1).wait()
            pltpu.async_copy(hp_hbm.at[src_all.at[t0 + 3]], rows1, sg1)
            pltpu.async_copy(dstp_hbm.at[brow + t0 + 3], dst1, sd1)

    pltpu.make_async_copy(rows0, acc_sh.at[dst0], ss0).wait()
    pltpu.make_async_copy(rows1, acc_sh.at[dst1], ss1).wait()
    plsc.subcore_barrier()

    @pl.when(s < _NS - 1)
    def _():
        pltpu.sync_copy(acc_sh.at[pl.ds(r0, _RPS)],
                        out_hbm.at[c, pl.ds(r0, _RPS)])

    @pl.when(s == _NS - 1)
    def _():
        pltpu.sync_copy(acc_sh.at[pl.ds(r0, _RPS_LAST)],
                        out_hbm.at[c, pl.ds(r0, _RPS_LAST)])


_sc_scatter = pl.kernel(
    _sc_scatter_body,
    out_type=jax.ShapeDtypeStruct((_NC, _N, _F), jnp.float32),
    mesh=_mesh,
    scratch_types=[
        pltpu.VMEM((_CH_PER_SUB, _CHUNK), jnp.int32),
        pltpu.VMEM((_CHUNK,), jnp.int32),
        pltpu.VMEM((_CHUNK,), jnp.int32),
        pltpu.VMEM((_CHUNK, _F), jnp.float32),
        pltpu.VMEM((_CHUNK, _F), jnp.float32),
        pltpu.VMEM_SHARED((_NPAD, _F), jnp.float32),
        pltpu.SemaphoreType.DMA,
        pltpu.SemaphoreType.DMA,
        pltpu.SemaphoreType.DMA,
        pltpu.SemaphoreType.DMA,
        pltpu.SemaphoreType.DMA,
        pltpu.SemaphoreType.DMA,
    ],
)


def _dinv_body(d0_ref, d1_ref, o_ref):
    deg = d0_ref[:, 0:1] + d1_ref[:, 0:1] - 1.0
    o_ref[...] = lax.rsqrt(deg)


_dinv_call = pl.pallas_call(
    _dinv_body,
    grid=(_NBLK,),
    in_specs=[pl.BlockSpec((_BLK, _F), lambda i: (i, 0)),
              pl.BlockSpec((_BLK, _F), lambda i: (i, 0))],
    out_specs=pl.BlockSpec((_BLK, 1), lambda i: (i, 0)),
    out_shape=jax.ShapeDtypeStruct((_N, 1), jnp.float32),
)


def _mm_scale_body(x_ref, w_ref, dinv_ref, o_ref):
    h = jnp.dot(x_ref[...], w_ref[...], preferred_element_type=jnp.float32)
    o_ref[...] = h * dinv_ref[...]


_mm_scale = pl.pallas_call(
    _mm_scale_body,
    grid=(_NBLK,),
    in_specs=[pl.BlockSpec((_BLK, _F), lambda i: (i, 0)),
              pl.BlockSpec((_F, _F), lambda i: (0, 0)),
              pl.BlockSpec((_BLK, 1), lambda i: (i, 0))],
    out_specs=pl.BlockSpec((_BLK, _F), lambda i: (i, 0)),
    out_shape=jax.ShapeDtypeStruct((_N, _F), jnp.float32),
)


def _comb_body(p0_ref, p1_ref, hp_ref, dinv_ref, y_ref, st_ref, acc):
    i = pl.program_id(0)

    @pl.when(i == 0)
    def _():
        acc[...] = jnp.zeros((8, _F), jnp.float32)

    y = (p0_ref[...] + p1_ref[...] - hp_ref[...]) * dinv_ref[...]
    y_ref[...] = y
    acc[0:1, :] += jnp.sum(y, axis=0, keepdims=True)
    acc[1:2, :] += jnp.sum(y * y, axis=0, keepdims=True)

    @pl.when(i == _NBLK - 1)
    def _():
        st_ref[...] = acc[...]


_comb = pl.pallas_call(
    _comb_body,
    grid=(_NBLK,),
    in_specs=[pl.BlockSpec((_BLK, _F), lambda i: (i, 0)),
              pl.BlockSpec((_BLK, _F), lambda i: (i, 0)),
              pl.BlockSpec((_BLK, _F), lambda i: (i, 0)),
              pl.BlockSpec((_BLK, 1), lambda i: (i, 0))],
    out_specs=[pl.BlockSpec((_BLK, _F), lambda i: (i, 0)),
               pl.BlockSpec((8, _F), lambda i: (0, 0))],
    out_shape=[jax.ShapeDtypeStruct((_N, _F), jnp.float32),
               jax.ShapeDtypeStruct((8, _F), jnp.float32)],
    scratch_shapes=[pltpu.VMEM((8, _F), jnp.float32)],
)


def _bn_block(y_ref, st_ref, g_ref, b_ref, prev_ref):
    mean = st_ref[0:1, :] * (1.0 / _N)
    var = st_ref[1:2, :] * (1.0 / _N) - mean * mean
    inv = lax.rsqrt(var + _EPS)
    z = (y_ref[...] - mean) * (inv * g_ref[...]) + b_ref[...]
    return jnp.maximum(z, 0.0) + prev_ref[...]


def _bn_mm_body(y_ref, st_ref, g_ref, b_ref, prev_ref, w_ref, dinv_ref,
                z_ref, hp_ref):
    z = _bn_block(y_ref, st_ref, g_ref, b_ref, prev_ref)
    z_ref[...] = z
    h = jnp.dot(z, w_ref[...], preferred_element_type=jnp.float32)
    hp_ref[...] = h * dinv_ref[...]


_bn_mm = pl.pallas_call(
    _bn_mm_body,
    grid=(_NBLK,),
    in_specs=[pl.BlockSpec((_BLK, _F), lambda i: (i, 0)),
              pl.BlockSpec((8, _F), lambda i: (0, 0)),
              pl.BlockSpec((1, _F), lambda i: (0, 0)),
              pl.BlockSpec((1, _F), lambda i: (0, 0)),
              pl.BlockSpec((_BLK, _F), lambda i: (i, 0)),
              pl.BlockSpec((_F, _F), lambda i: (0, 0)),
              pl.BlockSpec((_BLK, 1), lambda i: (i, 0))],
    out_specs=[pl.BlockSpec((_BLK, _F), lambda i: (i, 0)),
               pl.BlockSpec((_BLK, _F), lambda i: (i, 0))],
    out_shape=[jax.ShapeDtypeStruct((_N, _F), jnp.float32),
               jax.ShapeDtypeStruct((_N, _F), jnp.float32)],
)


def _bn_final_body(y_ref, st_ref, g_ref, b_ref, prev_ref, w_ref, bo_ref,
                   o_ref):
    z = _bn_block(y_ref, st_ref, g_ref, b_ref, prev_ref)
    h = jnp.dot(z, w_ref[...], preferred_element_type=jnp.float32)
    o_ref[...] = h + bo_ref[...]


_bn_final = pl.pallas_call(
    _bn_final_body,
    grid=(_NBLK,),
    in_specs=[pl.BlockSpec((_BLK, _F), lambda i: (i, 0)),
              pl.BlockSpec((8, _F), lambda i: (0, 0)),
              pl.BlockSpec((1, _F), lambda i: (0, 0)),
              pl.BlockSpec((1, _F), lambda i: (0, 0)),
              pl.BlockSpec((_BLK, _F), lambda i: (i, 0)),
              pl.BlockSpec((_F, _F), lambda i: (0, 0)),
              pl.BlockSpec((1, _F), lambda i: (0, 0))],
    out_specs=pl.BlockSpec((_BLK, _F), lambda i: (i, 0)),
    out_shape=jax.ShapeDtypeStruct((_N, _F), jnp.float32),
)


def kernel(x, edge_index, edge_attr, Ws, gammas, betas, W_out, b_out):
    del edge_attr
    x2 = x[0]
    src = edge_index[0]
    dst = edge_index[1]
    npad = _EPAD - _E
    pad_ar = jnp.arange(npad, dtype=jnp.int32)
    nchunks = _EPAD // _CHUNK
    srcp = jnp.concatenate([src, (pad_ar * 64) % _N]).reshape(nchunks, _CHUNK)
    dstp = jnp.concatenate(
        [dst, _N + (pad_ar % _NLAND)]).reshape(nchunks, _CHUNK)

    deg_table = jnp.ones((_N, _F), jnp.float32)
    deg_src = (jnp.arange(_EPAD, dtype=jnp.int32) % _N).reshape(
        nchunks, _CHUNK)
    degp = _sc_scatter(deg_table, deg_src, dstp)
    dinv = _dinv_call(degp[0], degp[1])

    prev = x2
    hp = _mm_scale(x2, Ws[0], dinv)
    out = None
    for i in range(_L):
        parts = _sc_scatter(hp, srcp, dstp)
        y, st = _comb(parts[0], parts[1], hp, dinv)
        g = gammas[i].reshape(1, _F)
        b = betas[i].reshape(1, _F)
        if i < _L - 1:
            prev, hp = _bn_mm(y, st, g, b, prev, Ws[i + 1], dinv)
        else:
            out = _bn_final(y, st, g, b, prev, W_out, b_out.reshape(1, _F))
    return out[None]


# speedup vs baseline: 1.2643x; 1.2643x over previous
import functools

import jax
import jax.numpy as jnp
from jax import lax
from jax.experimental import pallas as pl
from jax.experimental.pallas import tpu as pltpu
from jax.experimental.pallas import tpu_sc as plsc

_N = 10000
_E = 320000
_F = 128
_L = 3
_EPS = 1e-5

_NC = 2
_NS = 16
_CHUNK = 128
_CH_PER_SUB = 80
_CH_PER_CORE = _CH_PER_SUB * _NS
_EPAD = _CH_PER_CORE * _NC * _CHUNK
_NLAND = 128
_NPAD = _N + _NLAND
_RPS = 624
_RPS_LAST = _N - 15 * _RPS

_BLK = 1000
_NBLK = _N // _BLK

_mesh = plsc.VectorSubcoreMesh(core_axis_name="c", subcore_axis_name="s",
                               num_cores=_NC, num_subcores=_NS)


def _sc_scatter_body(hp_hbm, srcp_hbm, dstp_hbm, out_hbm,
                     src_all, dst0, dst1, rows0, rows1, acc_sh,
                     sg0, sg1, sd0, sd1):
    c = lax.axis_index("c")
    s = lax.axis_index("s")
    r0 = s * _RPS

    @pl.when(s < _NS - 1)
    def _():
        pltpu.sync_copy(hp_hbm.at[pl.ds(r0, _RPS)],
                        acc_sh.at[pl.ds(r0, _RPS)])

    @pl.when(s == _NS - 1)
    def _():
        pltpu.sync_copy(hp_hbm.at[pl.ds(r0, _RPS_LAST)],
                        acc_sh.at[pl.ds(r0, _RPS_LAST)])

    brow = c * _CH_PER_CORE + s * _CH_PER_SUB
    pltpu.sync_copy(srcp_hbm.at[pl.ds(brow, _CH_PER_SUB)], src_all)

    pltpu.async_copy(hp_hbm.at[src_all.at[0]], rows0, sg0)
    pltpu.async_copy(dstp_hbm.at[brow], dst0, sd0)
    plsc.subcore_barrier()

    @pl.loop(0, _CH_PER_SUB // 2)
    def _(i):
        t0 = 2 * i
        pltpu.async_copy(hp_hbm.at[src_all.at[t0 + 1]], rows1, sg1)
        pltpu.async_copy(dstp_hbm.at[brow + t0 + 1], dst1, sd1)
        pltpu.make_async_copy(hp_hbm.at[src_all.at[t0]], rows0, sg0).wait()
        pltpu.make_async_copy(dstp_hbm.at[brow + t0], dst0, sd0).wait()
        pltpu.sync_copy(rows0, acc_sh.at[dst0], add=True)

        @pl.when(i < _CH_PER_SUB // 2 - 1)
        def _():
            pltpu.async_copy(hp_hbm.at[src_all.at[t0 + 2]], rows0, sg0)
            pltpu.async_copy(dstp_hbm.at[brow + t0 + 2], dst0, sd0)

        pltpu.make_async_copy(hp_hbm.at[src_all.at[t0 + 1]], rows1,
                              sg1).wait()
        pltpu.make_async_copy(dstp_hbm.at[brow + t0 + 1], dst1, sd1).wait()
        pltpu.sync_copy(rows1, acc_sh.at[dst1], add=True)

    plsc.subcore_barrier()

    @pl.when(s < _NS - 1)
    def _():
        pltpu.sync_copy(acc_sh.at[pl.ds(r0, _RPS)],
                        out_hbm.at[c, pl.ds(r0, _RPS)])

    @pl.when(s == _NS - 1)
    def _():
        pltpu.sync_copy(acc_sh.at[pl.ds(r0, _RPS_LAST)],
                        out_hbm.at[c, pl.ds(r0, _RPS_LAST)])


_sc_scatter = pl.kernel(
    _sc_scatter_body,
    out_type=jax.ShapeDtypeStruct((_NC, _N, _F), jnp.float32),
    mesh=_mesh,
    scratch_types=[
        pltpu.VMEM((_CH_PER_SUB, _CHUNK), jnp.int32),
        pltpu.VMEM((_CHUNK,), jnp.int32),
        pltpu.VMEM((_CHUNK,), jnp.int32),
        pltpu.VMEM((_CHUNK, _F), jnp.float32),
        pltpu.VMEM((_CHUNK, _F), jnp.float32),
        pltpu.VMEM_SHARED((_NPAD, _F), jnp.float32),
        pltpu.SemaphoreType.DMA,
        pltpu.SemaphoreType.DMA,
        pltpu.SemaphoreType.DMA,
        pltpu.SemaphoreType.DMA,
    ],
)


def _dinv_body(d0_ref, d1_ref, o_ref):
    deg = d0_ref[:, 0:1] + d1_ref[:, 0:1] - 1.0
    o_ref[...] = lax.rsqrt(deg)


_dinv_call = pl.pallas_call(
    _dinv_body,
    grid=(_NBLK,),
    in_specs=[pl.BlockSpec((_BLK, _F), lambda i: (i, 0)),
              pl.BlockSpec((_BLK, _F), lambda i: (i, 0))],
    out_specs=pl.BlockSpec((_BLK, 1), lambda i: (i, 0)),
    out_shape=jax.ShapeDtypeStruct((_N, 1), jnp.float32),
)


def _mm_scale_body(x_ref, w_ref, dinv_ref, o_ref):
    h = jnp.dot(x_ref[...], w_ref[...], preferred_element_type=jnp.float32)
    o_ref[...] = h * dinv_ref[...]


_mm_scale = pl.pallas_call(
    _mm_scale_body,
    grid=(_NBLK,),
    in_specs=[pl.BlockSpec((_BLK, _F), lambda i: (i, 0)),
              pl.BlockSpec((_F, _F), lambda i: (0, 0)),
              pl.BlockSpec((_BLK, 1), lambda i: (i, 0))],
    out_specs=pl.BlockSpec((_BLK, _F), lambda i: (i, 0)),
    out_shape=jax.ShapeDtypeStruct((_N, _F), jnp.float32),
)


def _comb_body(p0_ref, p1_ref, hp_ref, dinv_ref, y_ref, st_ref, acc):
    i = pl.program_id(0)

    @pl.when(i == 0)
    def _():
        acc[...] = jnp.zeros((8, _F), jnp.float32)

    y = (p0_ref[...] + p1_ref[...] - hp_ref[...]) * dinv_ref[...]
    y_ref[...] = y
    acc[0:1, :] += jnp.sum(y, axis=0, keepdims=True)
    acc[1:2, :] += jnp.sum(y * y, axis=0, keepdims=True)

    @pl.when(i == _NBLK - 1)
    def _():
        st_ref[...] = acc[...]


_comb = pl.pallas_call(
    _comb_body,
    grid=(_NBLK,),
    in_specs=[pl.BlockSpec((_BLK, _F), lambda i: (i, 0)),
              pl.BlockSpec((_BLK, _F), lambda i: (i, 0)),
              pl.BlockSpec((_BLK, _F), lambda i: (i, 0)),
              pl.BlockSpec((_BLK, 1), lambda i: (i, 0))],
    out_specs=[pl.BlockSpec((_BLK, _F), lambda i: (i, 0)),
               pl.BlockSpec((8, _F), lambda i: (0, 0))],
    out_shape=[jax.ShapeDtypeStruct((_N, _F), jnp.float32),
               jax.ShapeDtypeStruct((8, _F), jnp.float32)],
    scratch_shapes=[pltpu.VMEM((8, _F), jnp.float32)],
)


def _bn_block(y_ref, st_ref, g_ref, b_ref, prev_ref):
    mean = st_ref[0:1, :] * (1.0 / _N)
    var = st_ref[1:2, :] * (1.0 / _N) - mean * mean
    inv = lax.rsqrt(var + _EPS)
    z = (y_ref[...] - mean) * (inv * g_ref[...]) + b_ref[...]
    return jnp.maximum(z, 0.0) + prev_ref[...]


def _bn_mm_body(y_ref, st_ref, g_ref, b_ref, prev_ref, w_ref, dinv_ref,
                z_ref, hp_ref):
    z = _bn_block(y_ref, st_ref, g_ref, b_ref, prev_ref)
    z_ref[...] = z
    h = jnp.dot(z, w_ref[...], preferred_element_type=jnp.float32)
    hp_ref[...] = h * dinv_ref[...]


_bn_mm = pl.pallas_call(
    _bn_mm_body,
    grid=(_NBLK,),
    in_specs=[pl.BlockSpec((_BLK, _F), lambda i: (i, 0)),
              pl.BlockSpec((8, _F), lambda i: (0, 0)),
              pl.BlockSpec((1, _F), lambda i: (0, 0)),
              pl.BlockSpec((1, _F), lambda i: (0, 0)),
              pl.BlockSpec((_BLK, _F), lambda i: (i, 0)),
              pl.BlockSpec((_F, _F), lambda i: (0, 0)),
              pl.BlockSpec((_BLK, 1), lambda i: (i, 0))],
    out_specs=[pl.BlockSpec((_BLK, _F), lambda i: (i, 0)),
               pl.BlockSpec((_BLK, _F), lambda i: (i, 0))],
    out_shape=[jax.ShapeDtypeStruct((_N, _F), jnp.float32),
               jax.ShapeDtypeStruct((_N, _F), jnp.float32)],
)


def _bn_final_body(y_ref, st_ref, g_ref, b_ref, prev_ref, w_ref, bo_ref,
                   o_ref):
    z = _bn_block(y_ref, st_ref, g_ref, b_ref, prev_ref)
    h = jnp.dot(z, w_ref[...], preferred_element_type=jnp.float32)
    o_ref[...] = h + bo_ref[...]


_bn_final = pl.pallas_call(
    _bn_final_body,
    grid=(_NBLK,),
    in_specs=[pl.BlockSpec((_BLK, _F), lambda i: (i, 0)),
              pl.BlockSpec((8, _F), lambda i: (0, 0)),
              pl.BlockSpec((1, _F), lambda i: (0, 0)),
              pl.BlockSpec((1, _F), lambda i: (0, 0)),
              pl.BlockSpec((_BLK, _F), lambda i: (i, 0)),
              pl.BlockSpec((_F, _F), lambda i: (0, 0)),
              pl.BlockSpec((1, _F), lambda i: (0, 0))],
    out_specs=pl.BlockSpec((_BLK, _F), lambda i: (i, 0)),
    out_shape=jax.ShapeDtypeStruct((_N, _F), jnp.float32),
)


def kernel(x, edge_index, edge_attr, Ws, gammas, betas, W_out, b_out):
    del edge_attr
    x2 = x[0]
    src = edge_index[0]
    dst = edge_index[1]
    npad = _EPAD - _E
    pad_ar = jnp.arange(npad, dtype=jnp.int32)
    nchunks = _EPAD // _CHUNK
    srcp = jnp.concatenate([src, (pad_ar * 64) % _N]).reshape(nchunks, _CHUNK)
    dstp = jnp.concatenate(
        [dst, _N + (pad_ar % _NLAND)]).reshape(nchunks, _CHUNK)

    deg_table = jnp.ones((_N, _F), jnp.float32)
    deg_src = (jnp.arange(_EPAD, dtype=jnp.int32) % _N).reshape(
        nchunks, _CHUNK)
    degp = _sc_scatter(deg_table, deg_src, dstp)
    dinv = _dinv_call(degp[0], degp[1])

    prev = x2
    hp = _mm_scale(x2, Ws[0], dinv)
    out = None
    for i in range(_L):
        parts = _sc_scatter(hp, srcp, dstp)
        y, st = _comb(parts[0], parts[1], hp, dinv)
        g = gammas[i].reshape(1, _F)
        b = betas[i].reshape(1, _F)
        if i < _L - 1:
            prev, hp = _bn_mm(y, st, g, b, prev, Ws[i + 1], dinv)
        else:
            out = _bn_final(y, st, g, b, prev, W_out, b_out.reshape(1, _F))
    return out[None]


# speedup vs baseline: 1.2900x; 1.0204x over previous
import functools

import jax
import jax.numpy as jnp
from jax import lax
from jax.experimental import pallas as pl
from jax.experimental.pallas import tpu as pltpu
from jax.experimental.pallas import tpu_sc as plsc

_N = 10000
_E = 320000
_F = 128
_L = 3
_EPS = 1e-5

_NC = 2
_NS = 16
_CHUNK = 128
_CH_PER_SUB = 80
_CH_PER_CORE = _CH_PER_SUB * _NS
_EPAD = _CH_PER_CORE * _NC * _CHUNK
_NLAND = 128
_NPAD = _N + _NLAND
_RPS = 624
_RPS_LAST = _N - 15 * _RPS

_BLK = 1000
_NBLK = _N // _BLK

_mesh = plsc.VectorSubcoreMesh(core_axis_name="c", subcore_axis_name="s",
                               num_cores=_NC, num_subcores=_NS)


def _sc_scatter_body(hp_hbm, srcp_hbm, dstp_hbm, out_hbm,
                     src_all, dst0, dst1, rows0, rows1, acc_sh,
                     sg0, sg1, sd0, sd1):
    c = lax.axis_index("c")
    s = lax.axis_index("s")
    r0 = s * _RPS

    @pl.when(s < _NS - 1)
    def _():
        pltpu.sync_copy(hp_hbm.at[pl.ds(r0, _RPS)],
                        acc_sh.at[pl.ds(r0, _RPS)])

    @pl.when(s == _NS - 1)
    def _():
        pltpu.sync_copy(hp_hbm.at[pl.ds(r0, _RPS_LAST)],
                        acc_sh.at[pl.ds(r0, _RPS_LAST)])

    brow = c * _CH_PER_CORE + s * _CH_PER_SUB
    pltpu.sync_copy(srcp_hbm.at[pl.ds(brow, _CH_PER_SUB)], src_all)

    pltpu.async_copy(hp_hbm.at[src_all.at[0]], rows0, sg0)
    pltpu.async_copy(dstp_hbm.at[brow], dst0, sd0)
    plsc.subcore_barrier()

    @pl.loop(0, _CH_PER_SUB // 2)
    def _(i):
        t0 = 2 * i
        pltpu.async_copy(hp_hbm.at[src_all.at[t0 + 1]], rows1, sg1)
        pltpu.async_copy(dstp_hbm.at[brow + t0 + 1], dst1, sd1)
        pltpu.make_async_copy(hp_hbm.at[src_all.at[t0]], rows0, sg0).wait()
        pltpu.make_async_copy(dstp_hbm.at[brow + t0], dst0, sd0).wait()
        pltpu.sync_copy(rows0, acc_sh.at[dst0], add=True)

        @pl.when(i < _CH_PER_SUB // 2 - 1)
        def _():
            pltpu.async_copy(hp_hbm.at[src_all.at[t0 + 2]], rows0, sg0)
            pltpu.async_copy(dstp_hbm.at[brow + t0 + 2], dst0, sd0)

        pltpu.make_async_copy(hp_hbm.at[src_all.at[t0 + 1]], rows1,
                              sg1).wait()
        pltpu.make_async_copy(dstp_hbm.at[brow + t0 + 1], dst1, sd1).wait()
        pltpu.sync_copy(rows1, acc_sh.at[dst1], add=True)

    plsc.subcore_barrier()

    @pl.when(s < _NS - 1)
    def _():
        pltpu.sync_copy(acc_sh.at[pl.ds(r0, _RPS)],
                        out_hbm.at[c, pl.ds(r0, _RPS)])

    @pl.when(s == _NS - 1)
    def _():
        pltpu.sync_copy(acc_sh.at[pl.ds(r0, _RPS_LAST)],
                        out_hbm.at[c, pl.ds(r0, _RPS_LAST)])


_sc_scatter = pl.kernel(
    _sc_scatter_body,
    out_type=jax.ShapeDtypeStruct((_NC, _N, _F), jnp.float32),
    mesh=_mesh,
    scratch_types=[
        pltpu.VMEM((_CH_PER_SUB, _CHUNK), jnp.int32),
        pltpu.VMEM((_CHUNK,), jnp.int32),
        pltpu.VMEM((_CHUNK,), jnp.int32),
        pltpu.VMEM((_CHUNK, _F), jnp.float32),
        pltpu.VMEM((_CHUNK, _F), jnp.float32),
        pltpu.VMEM_SHARED((_NPAD, _F), jnp.float32),
        pltpu.SemaphoreType.DMA,
        pltpu.SemaphoreType.DMA,
        pltpu.SemaphoreType.DMA,
        pltpu.SemaphoreType.DMA,
    ],
)


def _mm_raw_body(x_ref, w_ref, o_ref):
    o_ref[...] = jnp.dot(x_ref[...], w_ref[...],
                         preferred_element_type=jnp.float32)


_mm_raw = pl.pallas_call(
    _mm_raw_body,
    grid=(_NBLK,),
    in_specs=[pl.BlockSpec((_BLK, _F), lambda i: (i, 0)),
              pl.BlockSpec((_F, _F), lambda i: (0, 0))],
    out_specs=pl.BlockSpec((_BLK, _F), lambda i: (i, 0)),
    out_shape=jax.ShapeDtypeStruct((_N, _F), jnp.float32),
)


def _prep_body(h_ref, d0_ref, d1_ref, hp_ref, dinv_ref):
    deg = d0_ref[:, 0:1] + d1_ref[:, 0:1] - 1.0
    dinv = lax.rsqrt(deg)
    dinv_ref[...] = dinv
    hp_ref[...] = h_ref[...] * dinv


_prep = pl.pallas_call(
    _prep_body,
    grid=(_NBLK,),
    in_specs=[pl.BlockSpec((_BLK, _F), lambda i: (i, 0)),
              pl.BlockSpec((_BLK, _F), lambda i: (i, 0)),
              pl.BlockSpec((_BLK, _F), lambda i: (i, 0))],
    out_specs=[pl.BlockSpec((_BLK, _F), lambda i: (i, 0)),
               pl.BlockSpec((_BLK, 1), lambda i: (i, 0))],
    out_shape=[jax.ShapeDtypeStruct((_N, _F), jnp.float32),
               jax.ShapeDtypeStruct((_N, 1), jnp.float32)],
)


def _stage_phase0(p0, p1, hp, dinv, y_scr, acc, i):
    @pl.when(i == 0)
    def _():
        acc[...] = jnp.zeros((8, _F), jnp.float32)

    y = (p0[...] + p1[...] - hp[...]) * dinv[...]
    y_scr[pl.ds(i * _BLK, _BLK), :] = y
    acc[0:1, :] += jnp.sum(y, axis=0, keepdims=True)
    acc[1:2, :] += jnp.sum(y * y, axis=0, keepdims=True)


def _stage_z(y_scr, acc, g, b, prev, i):
    y = y_scr[pl.ds(i * _BLK, _BLK), :]
    mean = acc[0:1, :] * (1.0 / _N)
    var = acc[1:2, :] * (1.0 / _N) - mean * mean
    inv = lax.rsqrt(var + _EPS)
    z = (y - mean) * (inv * g[...]) + b[...]
    return jnp.maximum(z, 0.0) + prev[...]


def _stage_mid_body(p0, p1, hp, dinv, g, b, prev, w,
                    z_ref, hpn_ref, y_scr, acc):
    ph = pl.program_id(0)
    i = pl.program_id(1)

    @pl.when(ph == 0)
    def _():
        _stage_phase0(p0, p1, hp, dinv, y_scr, acc, i)

    @pl.when(ph == 1)
    def _():
        z = _stage_z(y_scr, acc, g, b, prev, i)
        z_ref[...] = z
        h = jnp.dot(z, w[...], preferred_element_type=jnp.float32)
        hpn_ref[...] = h * dinv[...]


_stage_mid = pl.pallas_call(
    _stage_mid_body,
    grid=(2, _NBLK),
    in_specs=[pl.BlockSpec((_BLK, _F), lambda ph, i: (i * (1 - ph), 0)),
              pl.BlockSpec((_BLK, _F), lambda ph, i: (i * (1 - ph), 0)),
              pl.BlockSpec((_BLK, _F), lambda ph, i: (i * (1 - ph), 0)),
              pl.BlockSpec((_BLK, 1), lambda ph, i: (i, 0)),
              pl.BlockSpec((1, _F), lambda ph, i: (0, 0)),
              pl.BlockSpec((1, _F), lambda ph, i: (0, 0)),
              pl.BlockSpec((_BLK, _F), lambda ph, i: (i * ph, 0)),
              pl.BlockSpec((_F, _F), lambda ph, i: (0, 0))],
    out_specs=[pl.BlockSpec((_BLK, _F), lambda ph, i: (i * ph, 0)),
               pl.BlockSpec((_BLK, _F), lambda ph, i: (i * ph, 0))],
    out_shape=[jax.ShapeDtypeStruct((_N, _F), jnp.float32),
               jax.ShapeDtypeStruct((_N, _F), jnp.float32)],
    scratch_shapes=[pltpu.VMEM((_N, _F), jnp.float32),
                    pltpu.VMEM((8, _F), jnp.float32)],
)


def _stage_fin_body(p0, p1, hp, dinv, g, b, prev, w, bo,
                    o_ref, y_scr, acc):
    ph = pl.program_id(0)
    i = pl.program_id(1)

    @pl.when(ph == 0)
    def _():
        _stage_phase0(p0, p1, hp, dinv, y_scr, acc, i)

    @pl.when(ph == 1)
    def _():
        z = _stage_z(y_scr, acc, g, b, prev, i)
        h = jnp.dot(z, w[...], preferred_element_type=jnp.float32)
        o_ref[...] = h + bo[...]


_stage_fin = pl.pallas_call(
    _stage_fin_body,
    grid=(2, _NBLK),
    in_specs=[pl.BlockSpec((_BLK, _F), lambda ph, i: (i * (1 - ph), 0)),
              pl.BlockSpec((_BLK, _F), lambda ph, i: (i * (1 - ph), 0)),
              pl.BlockSpec((_BLK, _F), lambda ph, i: (i * (1 - ph), 0)),
              pl.BlockSpec((_BLK, 1), lambda ph, i: (i, 0)),
              pl.BlockSpec((1, _F), lambda ph, i: (0, 0)),
              pl.BlockSpec((1, _F), lambda ph, i: (0, 0)),
              pl.BlockSpec((_BLK, _F), lambda ph, i: (i * ph, 0)),
              pl.BlockSpec((_F, _F), lambda ph, i: (0, 0)),
              pl.BlockSpec((1, _F), lambda ph, i: (0, 0))],
    out_specs=pl.BlockSpec((_BLK, _F), lambda ph, i: (i * ph, 0)),
    out_shape=jax.ShapeDtypeStruct((_N, _F), jnp.float32),
    scratch_shapes=[pltpu.VMEM((_N, _F), jnp.float32),
                    pltpu.VMEM((8, _F), jnp.float32)],
)


def kernel(x, edge_index, edge_attr, Ws, gammas, betas, W_out, b_out):
    del edge_attr
    x2 = x[0]
    src = edge_index[0]
    dst = edge_index[1]
    npad = _EPAD - _E
    pad_ar = jnp.arange(npad, dtype=jnp.int32)
    nchunks = _EPAD // _CHUNK
    srcp = jnp.concatenate([src, (pad_ar * 64) % _N]).reshape(nchunks, _CHUNK)
    dstp = jnp.concatenate(
        [dst, _N + (pad_ar % _NLAND)]).reshape(nchunks, _CHUNK)

    deg_table = jnp.ones((_N, _F), jnp.float32)
    deg_src = (jnp.arange(_EPAD, dtype=jnp.int32) % _N).reshape(
        nchunks, _CHUNK)
    degp = _sc_scatter(deg_table, deg_src, dstp)
    h1 = _mm_raw(x2, Ws[0])
    hp, dinv = _prep(h1, degp[0], degp[1])

    prev = x2
    out = None
    for i in range(_L):
        parts = _sc_scatter(hp, srcp, dstp)
        g = gammas[i].reshape(1, _F)
        b = betas[i].reshape(1, _F)
        if i < _L - 1:
            prev, hp = _stage_mid(parts[0], parts[1], hp, dinv, g, b, prev,
                                  Ws[i + 1])
        else:
            out = _stage_fin(parts[0], parts[1], hp, dinv, g, b, prev,
                             W_out, b_out.reshape(1, _F))
    return out[None]


# speedup vs baseline: 1.2913x; 1.0010x over previous
import jax
import jax.numpy as jnp
from jax import lax
from jax.experimental import pallas as pl
from jax.experimental.pallas import tpu as pltpu
from jax.experimental.pallas import tpu_sc as plsc

_N = 10000
_E = 320000
_F = 128
_L = 3
_EPS = 1e-5

_NC = 2
_NS = 16
_CHUNK = 128
_CH_PER_SUB = 80
_CH_PER_CORE = _CH_PER_SUB * _NS
_EPAD = _CH_PER_CORE * _NC * _CHUNK
_NLAND = 128
_NPAD = _N + _NLAND
_RPS = 624
_RPS_LAST = _N - 15 * _RPS

_BLK = 1000
_NBLK = _N // _BLK

_mesh = plsc.VectorSubcoreMesh(core_axis_name="c", subcore_axis_name="s",
                               num_cores=_NC, num_subcores=_NS)


def _sc_scatter_body(hp_hbm, srcp_hbm, dstp_hbm, out_hbm,
                     src_all, dst0, dst1, rows0, rows1, acc_sh,
                     sg0, sg1, sd0, sd1):
    c = lax.axis_index("c")
    s = lax.axis_index("s")
    r0 = s * _RPS

    @pl.when(s < _NS - 1)
    def _():
        pltpu.sync_copy(hp_hbm.at[pl.ds(r0, _RPS)],
                        acc_sh.at[pl.ds(r0, _RPS)])

    @pl.when(s == _NS - 1)
    def _():
        pltpu.sync_copy(hp_hbm.at[pl.ds(r0, _RPS_LAST)],
                        acc_sh.at[pl.ds(r0, _RPS_LAST)])

    brow = c * _CH_PER_CORE + s * _CH_PER_SUB
    pltpu.sync_copy(srcp_hbm.at[pl.ds(brow, _CH_PER_SUB)], src_all)

    pltpu.async_copy(hp_hbm.at[src_all.at[0]], rows0, sg0)
    pltpu.async_copy(dstp_hbm.at[brow], dst0, sd0)
    plsc.subcore_barrier()

    @pl.loop(0, _CH_PER_SUB // 2)
    def _(i):
        t0 = 2 * i
        pltpu.async_copy(hp_hbm.at[src_all.at[t0 + 1]], rows1, sg1)
        pltpu.async_copy(dstp_hbm.at[brow + t0 + 1], dst1, sd1)
        pltpu.make_async_copy(hp_hbm.at[src_all.at[t0]], rows0, sg0).wait()
        pltpu.make_async_copy(dstp_hbm.at[brow + t0], dst0, sd0).wait()
        pltpu.sync_copy(rows0, acc_sh.at[dst0], add=True)

        @pl.when(i < _CH_PER_SUB // 2 - 1)
        def _():
            pltpu.async_copy(hp_hbm.at[src_all.at[t0 + 2]], rows0, sg0)
            pltpu.async_copy(dstp_hbm.at[brow + t0 + 2], dst0, sd0)

        pltpu.make_async_copy(hp_hbm.at[src_all.at[t0 + 1]], rows1,
                              sg1).wait()
        pltpu.make_async_copy(dstp_hbm.at[brow + t0 + 1], dst1, sd1).wait()
        pltpu.sync_copy(rows1, acc_sh.at[dst1], add=True)

    plsc.subcore_barrier()

    @pl.when(s < _NS - 1)
    def _():
        pltpu.sync_copy(acc_sh.at[pl.ds(r0, _RPS)],
                        out_hbm.at[c, pl.ds(r0, _RPS)])

    @pl.when(s == _NS - 1)
    def _():
        pltpu.sync_copy(acc_sh.at[pl.ds(r0, _RPS_LAST)],
                        out_hbm.at[c, pl.ds(r0, _RPS_LAST)])


_sc_scatter = pl.kernel(
    _sc_scatter_body,
    out_type=jax.ShapeDtypeStruct((_NC, _N, _F), jnp.float32),
    mesh=_mesh,
    scratch_types=[
        pltpu.VMEM((_CH_PER_SUB, _CHUNK), jnp.int32),
        pltpu.VMEM((_CHUNK,), jnp.int32),
        pltpu.VMEM((_CHUNK,), jnp.int32),
        pltpu.VMEM((_CHUNK, _F), jnp.float32),
        pltpu.VMEM((_CHUNK, _F), jnp.float32),
        pltpu.VMEM_SHARED((_NPAD, _F), jnp.float32),
        pltpu.SemaphoreType.DMA,
        pltpu.SemaphoreType.DMA,
        pltpu.SemaphoreType.DMA,
        pltpu.SemaphoreType.DMA,
    ],
)


def _mm_raw_body(x_ref, w_ref, o_ref):
    o_ref[...] = jnp.dot(x_ref[...], w_ref[...],
                         preferred_element_type=jnp.float32)


_mm_raw = pl.pallas_call(
    _mm_raw_body,
    grid=(_NBLK,),
    in_specs=[pl.BlockSpec((_BLK, _F), lambda i: (i, 0)),
              pl.BlockSpec((_F, _F), lambda i: (0, 0))],
    out_specs=pl.BlockSpec((_BLK, _F), lambda i: (i, 0)),
    out_shape=jax.ShapeDtypeStruct((_N, _F), jnp.float32),
)


def _prep_body(h_ref, d0_ref, d1_ref, hp_ref, dinv_ref):
    deg = d0_ref[:, 0:1] + d1_ref[:, 0:1] - 1.0
    dinv = lax.rsqrt(deg)
    dinv_ref[...] = dinv
    hp_ref[...] = h_ref[...] * dinv


_prep = pl.pallas_call(
    _prep_body,
    grid=(_NBLK,),
    in_specs=[pl.BlockSpec((_BLK, _F), lambda i: (i, 0)),
              pl.BlockSpec((_BLK, _F), lambda i: (i, 0)),
              pl.BlockSpec((_BLK, _F), lambda i: (i, 0))],
    out_specs=[pl.BlockSpec((_BLK, _F), lambda i: (i, 0)),
               pl.BlockSpec((_BLK, 1), lambda i: (i, 0))],
    out_shape=[jax.ShapeDtypeStruct((_N, _F), jnp.float32),
               jax.ShapeDtypeStruct((_N, 1), jnp.float32)],
)


def _stage_phase0(p0, p1, hp, dinv, y_scr, acc, i):
    @pl.when(i == 0)
    def _():
        acc[...] = jnp.zeros((8, _F), jnp.float32)

    y = (p0[...] + p1[...] - hp[...]) * dinv[...]
    y_scr[pl.ds(i * _BLK, _BLK), :] = y
    acc[0:1, :] += jnp.sum(y, axis=0, keepdims=True)
    acc[1:2, :] += jnp.sum(y * y, axis=0, keepdims=True)


def _stage_z(y_scr, acc, g, b, prev, i):
    y = y_scr[pl.ds(i * _BLK, _BLK), :]
    mean = acc[0:1, :] * (1.0 / _N)
    var = acc[1:2, :] * (1.0 / _N) - mean * mean
    inv = lax.rsqrt(var + _EPS)
    z = (y - mean) * (inv * g[...]) + b[...]
    return jnp.maximum(z, 0.0) + prev[...]


def _stage_mid_body(p0, p1, hp, dinv, g, b, prev, w,
                    z_ref, hpn_ref, y_scr, acc):
    ph = pl.program_id(0)
    i = pl.program_id(1)

    @pl.when(ph == 0)
    def _():
        _stage_phase0(p0, p1, hp, dinv, y_scr, acc, i)

    @pl.when(ph == 1)
    def _():
        z = _stage_z(y_scr, acc, g, b, prev, i)
        z_ref[...] = z
        h = jnp.dot(z, w[...], preferred_element_type=jnp.float32)
        hpn_ref[...] = h * dinv[...]


_stage_mid = pl.pallas_call(
    _stage_mid_body,
    grid=(2, _NBLK),
    in_specs=[pl.BlockSpec((_BLK, _F), lambda ph, i: (i * (1 - ph), 0)),
              pl.BlockSpec((_BLK, _F), lambda ph, i: (i * (1 - ph), 0)),
              pl.BlockSpec((_BLK, _F), lambda ph, i: (i * (1 - ph), 0)),
              pl.BlockSpec((_BLK, 1), lambda ph, i: (i, 0)),
              pl.BlockSpec((1, _F), lambda ph, i: (0, 0)),
              pl.BlockSpec((1, _F), lambda ph, i: (0, 0)),
              pl.BlockSpec((_BLK, _F), lambda ph, i: (i * ph, 0)),
              pl.BlockSpec((_F, _F), lambda ph, i: (0, 0))],
    out_specs=[pl.BlockSpec((_BLK, _F), lambda ph, i: (i * ph, 0)),
               pl.BlockSpec((_BLK, _F), lambda ph, i: (i * ph, 0))],
    out_shape=[jax.ShapeDtypeStruct((_N, _F), jnp.float32),
               jax.ShapeDtypeStruct((_N, _F), jnp.float32)],
    scratch_shapes=[pltpu.VMEM((_N, _F), jnp.float32),
                    pltpu.VMEM((8, _F), jnp.float32)],
)


def _stage_fin_body(p0, p1, hp, dinv, g, b, prev, w, bo,
                    o_ref, y_scr, acc):
    ph = pl.program_id(0)
    i = pl.program_id(1)

    @pl.when(ph == 0)
    def _():
        _stage_phase0(p0, p1, hp, dinv, y_scr, acc, i)

    @pl.when(ph == 1)
    def _():
        z = _stage_z(y_scr, acc, g, b, prev, i)
        h = jnp.dot(z, w[...], preferred_element_type=jnp.float32)
        o_ref[...] = h + bo[...]


_stage_fin = pl.pallas_call(
    _stage_fin_body,
    grid=(2, _NBLK),
    in_specs=[pl.BlockSpec((_BLK, _F), lambda ph, i: (i * (1 - ph), 0)),
              pl.BlockSpec((_BLK, _F), lambda ph, i: (i * (1 - ph), 0)),
              pl.BlockSpec((_BLK, _F), lambda ph, i: (i * (1 - ph), 0)),
              pl.BlockSpec((_BLK, 1), lambda ph, i: (i, 0)),
              pl.BlockSpec((1, _F), lambda ph, i: (0, 0)),
              pl.BlockSpec((1, _F), lambda ph, i: (0, 0)),
              pl.BlockSpec((_BLK, _F), lambda ph, i: (i * ph, 0)),
              pl.BlockSpec((_F, _F), lambda ph, i: (0, 0)),
              pl.BlockSpec((1, _F), lambda ph, i: (0, 0))],
    out_specs=pl.BlockSpec((_BLK, _F), lambda ph, i: (i * ph, 0)),
    out_shape=jax.ShapeDtypeStruct((_N, _F), jnp.float32),
    scratch_shapes=[pltpu.VMEM((_N, _F), jnp.float32),
                    pltpu.VMEM((8, _F), jnp.float32)],
)


def kernel(x, edge_index, edge_attr, Ws, gammas, betas, W_out, b_out):
    del edge_attr
    x2 = x[0]
    src = edge_index[0]
    dst = edge_index[1]
    npad = _EPAD - _E
    pad_ar = jnp.arange(npad, dtype=jnp.int32)
    nchunks = _EPAD // _CHUNK
    srcp = jnp.concatenate([src, (pad_ar * 64) % _N]).reshape(nchunks, _CHUNK)
    dstp = jnp.concatenate(
        [dst, _N + (pad_ar % _NLAND)]).reshape(nchunks, _CHUNK)

    deg_table = jnp.ones((_N, _F), jnp.float32)
    deg_src = (jnp.arange(_EPAD, dtype=jnp.int32) % _N).reshape(
        nchunks, _CHUNK)
    degp = _sc_scatter(deg_table, deg_src, dstp)
    h1 = _mm_raw(x2, Ws[0])
    hp, dinv = _prep(h1, degp[0], degp[1])

    prev = x2
    out = None
    for i in range(_L):
        parts = _sc_scatter(hp, srcp, dstp)
        g = gammas[i].reshape(1, _F)
        b = betas[i].reshape(1, _F)
        if i < _L - 1:
            prev, hp = _stage_mid(parts[0], parts[1], hp, dinv, g, b, prev,
                                  Ws[i + 1])
        else:
            out = _stage_fin(parts[0], parts[1], hp, dinv, g, b, prev,
                             W_out, b_out.reshape(1, _F))
    return out[None]
